# baseline (speedup 1.0000x reference)
import functools

import jax
import jax.numpy as jnp
from jax import lax
from jax.experimental import pallas as pl
from jax.experimental.pallas import tpu as pltpu
from jax.experimental.pallas import tpu_sc as plsc

N = 3200000
NC = 2
NS = 16
NW = NC * NS
CHUNK = N // NW
LANES = 16
NSUB = 5
SUB = CHUNK // NSUB
UNROLL = 5
SITERS = SUB // (UNROLL * LANES)

_mesh = plsc.VectorSubcoreMesh(core_axis_name="c", subcore_axis_name="s")


@functools.partial(
    pl.kernel,
    out_type=jax.ShapeDtypeStruct((NW, LANES), jnp.float32),
    mesh=_mesh,
    scratch_types=[
        [pltpu.VMEM((SUB,), jnp.float32) for _ in range(NSUB)],
        pltpu.VMEM((LANES,), jnp.float32),
        [pltpu.SemaphoreType.DMA for _ in range(NSUB)],
    ],
)
def _partial_sums(x_hbm, out_hbm, bufs, part, sems):
    wid = lax.axis_index("s") * NC + lax.axis_index("c")
    base = wid * CHUNK

    copies = [
        pltpu.make_async_copy(
            x_hbm.at[pl.ds(base + k * SUB, SUB)], bufs[k], sems[k]
        )
        for k in range(NSUB)
    ]
    for k in range(NSUB):
        copies[k].start()

    total = jnp.zeros((LANES,), jnp.float32)
    for k in range(NSUB):
        copies[k].wait()

        def body(i, accs, buf=bufs[k]):
            off = i * (UNROLL * LANES)
            return tuple(
                accs[j] + buf[pl.ds(off + j * LANES, LANES)]
                for j in range(UNROLL)
            )

        zero = jnp.zeros((LANES,), jnp.float32)
        accs = lax.fori_loop(0, SITERS, body, (zero,) * UNROLL)
        for j in range(UNROLL):
            total = total + accs[j]

    part[...] = total
    pltpu.sync_copy(part, out_hbm.at[wid])


@functools.partial(
    pl.kernel,
    out_type=jax.ShapeDtypeStruct((1, 1), jnp.float32),
    mesh=_mesh,
    scratch_types=[
        pltpu.VMEM((NW, LANES), jnp.float32),
        pltpu.VMEM((LANES,), jnp.float32),
    ],
)
def _combine(parts_hbm, out_hbm, buf, res):
    c = lax.axis_index("c")
    s = lax.axis_index("s")

    @pl.when(jnp.logical_and(c == 0, s == 0))
    def _():
        pltpu.sync_copy(parts_hbm, buf)
        total = buf[0, :]
        for i in range(1, NW):
            total = total + buf[i, :]
        scalar = total[0]
        for i in range(1, LANES):
            scalar = scalar + total[i]
        res[...] = jnp.full((LANES,), scalar, jnp.float32)
        pltpu.sync_copy(res.at[pl.ds(0, 1)], out_hbm.at[0])


def kernel(atomic_energy):
    x = atomic_energy.reshape(-1)
    parts = _partial_sums(x)
    return _combine(parts)


# speedup vs baseline: 1.1169x; 1.1169x over previous
import functools

import jax
import jax.numpy as jnp
from jax import lax
from jax.experimental import pallas as pl
from jax.experimental.pallas import tpu as pltpu
from jax.experimental.pallas import tpu_sc as plsc

N = 3200000
NC = 2
NS = 16
NW = NC * NS
CHUNK = N // NW
LANES = 16
NSUB = 5
SUB = CHUNK // NSUB
UNROLL = 5
SITERS = SUB // (UNROLL * LANES)

_mesh = plsc.VectorSubcoreMesh(core_axis_name="c", subcore_axis_name="s")


@functools.partial(
    pl.kernel,
    out_type=jax.ShapeDtypeStruct((NW, LANES), jnp.float32),
    mesh=_mesh,
    scratch_types=[
        [pltpu.VMEM((SUB,), jnp.float32) for _ in range(NSUB)],
        pltpu.VMEM((LANES,), jnp.float32),
        [pltpu.SemaphoreType.DMA for _ in range(NSUB)],
    ],
)
def _partial_sums(x_hbm, out_hbm, bufs, part, sems):
    wid = lax.axis_index("s") * NC + lax.axis_index("c")
    base = wid * CHUNK

    copies = [
        pltpu.make_async_copy(
            x_hbm.at[pl.ds(base + k * SUB, SUB)], bufs[k], sems[k]
        )
        for k in range(NSUB)
    ]
    for k in range(NSUB):
        copies[k].start()

    total = jnp.zeros((LANES,), jnp.float32)
    for k in range(NSUB):
        copies[k].wait()

        def body(i, accs, buf=bufs[k]):
            off = i * (UNROLL * LANES)
            return tuple(
                accs[j] + buf[pl.ds(off + j * LANES, LANES)]
                for j in range(UNROLL)
            )

        zero = jnp.zeros((LANES,), jnp.float32)
        accs = lax.fori_loop(0, SITERS, body, (zero,) * UNROLL)
        for j in range(UNROLL):
            total = total + accs[j]

    part[...] = total
    pltpu.sync_copy(part, out_hbm.at[wid])


def _combine_body(parts_ref, out_ref):
    out_ref[...] = jnp.sum(parts_ref[...]).reshape(1, 1)


_combine = pl.pallas_call(
    _combine_body,
    out_shape=jax.ShapeDtypeStruct((1, 1), jnp.float32),
)


def kernel(atomic_energy):
    x = atomic_energy.reshape(-1)
    parts = _partial_sums(x)
    return _combine(parts)
